# baseline (speedup 1.0000x reference)
import functools

import jax
import jax.numpy as jnp
from jax import lax
from jax.experimental import pallas as pl
from jax.experimental.pallas import tpu as pltpu
from jax.experimental.pallas import tpu_sc as plsc

S, D, F, E, K = 2048, 1024, 4096, 8, 2
NA = S * K
BM = 256
NB = (NA + E * (BM - 1) + BM - 1) // BM
NR = NB * BM
FSPLIT = 2
FB = F // FSPLIT

_NC, _NS = 2, 16
_NW = _NC * _NS
_RPW = NR // _NW
_GCH = 64
_TPW = S // _NW
_TCH = 32


def _router_body(x_ref, wr_ref, br_ref, idx_ref, w_ref):
    x = x_ref[...]
    logits = lax.dot_general(x, wr_ref[...], (((1,), (0,)), ((), ())),
                             preferred_element_type=jnp.float32)
    logits = logits + br_ref[...]
    ids = lax.broadcasted_iota(jnp.int32, (S, E), 1)
    v1 = jnp.max(logits, axis=1, keepdims=True)
    i1 = jnp.min(jnp.where(logits == v1, ids, E), axis=1, keepdims=True)
    neg = jnp.float32(-3.4e38)
    l2 = jnp.where(ids == i1, neg, logits)
    v2 = jnp.max(l2, axis=1, keepdims=True)
    i2 = jnp.min(jnp.where(l2 == v2, ids, E), axis=1, keepdims=True)
    t = jnp.exp(v2 - v1)
    w1 = 1.0 / (1.0 + t)
    w2 = t / (1.0 + t)
    lane = lax.broadcasted_iota(jnp.int32, (S, K), 1)
    idx_ref[...] = jnp.where(lane == 0, i1, i2)
    w_ref[...] = jnp.where(lane == 0, w1, w2)


def _router(x, Wr, br2):
    return pl.pallas_call(
        _router_body,
        out_shape=(jax.ShapeDtypeStruct((S, K), jnp.int32),
                   jax.ShapeDtypeStruct((S, K), jnp.float32)),
    )(x, Wr, br2)


def _sc_dispatch(xp, pos_e, pos_o):
    mesh = plsc.VectorSubcoreMesh(core_axis_name="c", subcore_axis_name="s")

    @functools.partial(
        pl.kernel,
        out_type=jax.ShapeDtypeStruct((NR, D // 2), jnp.int32),
        mesh=mesh,
        scratch_types=[pltpu.VMEM((_TPW,), jnp.int32),
                       pltpu.VMEM((_TPW,), jnp.int32),
                       pltpu.VMEM((_TPW, D // 2), jnp.int32),
                       pltpu.SemaphoreType.DMA],
    )
    def k(x_hbm, pe_hbm, po_hbm, out_hbm, idxe_v, idxo_v, rows_v, sem):
        wid = lax.axis_index("s") * _NC + lax.axis_index("c")
        t0 = wid * _TPW
        pltpu.sync_copy(pe_hbm.at[pl.ds(t0, _TPW)], idxe_v)
        pltpu.sync_copy(po_hbm.at[pl.ds(t0, _TPW)], idxo_v)
        pltpu.sync_copy(x_hbm.at[pl.ds(t0, _TPW)], rows_v)
        cp_e = pltpu.async_copy(rows_v, out_hbm.at[idxe_v], sem)
        cp_o = pltpu.async_copy(rows_v, out_hbm.at[idxo_v], sem)
        cp_e.wait()
        cp_o.wait()

    return k(xp, pos_e, pos_o)


def _ffn_body(s_ref, xs_ref, w1_ref, b1_ref, w2_ref, b2_ref, y_ref):
    b = pl.program_id(0)
    f = pl.program_id(1)

    @pl.when(b < s_ref[0])
    def _():
        xb = xs_ref[...].astype(jnp.float32)
        h = lax.dot_general(xb, w1_ref[0], (((1,), (0,)), ((), ())),
                            preferred_element_type=jnp.float32)
        h = jnp.maximum(h + b1_ref[0, 0], 0.0)
        y = lax.dot_general(h, w2_ref[0], (((1,), (0,)), ((), ())),
                            preferred_element_type=jnp.float32)

        @pl.when(f == 0)
        def _():
            y_ref[...] = y + b2_ref[0]

        @pl.when(f != 0)
        def _():
            y_ref[...] += y


def _ffn(sp, xs, W1, b1, W2, b2):
    grid_spec = pltpu.PrefetchScalarGridSpec(
        num_scalar_prefetch=1,
        grid=(NB, FSPLIT),
        in_specs=[
            pl.BlockSpec((BM, D), lambda b, f, s: (b, 0)),
            pl.BlockSpec((1, D, FB), lambda b, f, s: (s[1 + b], 0, f)),
            pl.BlockSpec((1, 1, 1, FB), lambda b, f, s: (s[1 + b], f, 0, 0)),
            pl.BlockSpec((1, FB, D), lambda b, f, s: (s[1 + b], f, 0)),
            pl.BlockSpec((1, 1, D), lambda b, f, s: (s[1 + b], 0, 0)),
        ],
        out_specs=pl.BlockSpec((BM, D), lambda b, f, s: (b, 0)),
    )
    return pl.pallas_call(
        _ffn_body,
        grid_spec=grid_spec,
        out_shape=jax.ShapeDtypeStruct((NR, D), jnp.float32),
        compiler_params=pltpu.CompilerParams(
            dimension_semantics=("arbitrary", "arbitrary")),
    )(sp, xs, W1, b1.reshape(E, FSPLIT, 1, FB), W2, b2.reshape(E, 1, D))


def _sc_combine(y, posflat, we16, wo16):
    mesh = plsc.VectorSubcoreMesh(core_axis_name="c", subcore_axis_name="s")

    @functools.partial(
        pl.kernel,
        out_type=jax.ShapeDtypeStruct((S, D), jnp.float32),
        mesh=mesh,
        scratch_types=[pltpu.VMEM((K * _TCH,), jnp.int32),
                       pltpu.VMEM((_TCH, 16), jnp.float32),
                       pltpu.VMEM((_TCH, 16), jnp.float32),
                       pltpu.VMEM((K * _TCH, D), jnp.float32),
                       pltpu.VMEM((_TCH, D), jnp.float32),
                       pltpu.SemaphoreType.DMA],
    )
    def k(y_hbm, pos_hbm, we_hbm, wo_hbm, out_hbm,
          idx_v, we_v, wo_v, rows_v, out_v, sem):
        wid = lax.axis_index("s") * _NC + lax.axis_index("c")
        tbase = wid * _TPW

        def chunk(c, carry):
            t0 = tbase + c * _TCH
            pltpu.sync_copy(pos_hbm.at[pl.ds(t0 * K, K * _TCH)], idx_v)
            pltpu.sync_copy(we_hbm.at[pl.ds(t0, _TCH)], we_v)
            pltpu.sync_copy(wo_hbm.at[pl.ds(t0, _TCH)], wo_v)
            pltpu.async_copy(y_hbm.at[idx_v], rows_v, sem).wait()

            def tok(i, carry2):
                w0 = we_v[i, :]
                w1 = wo_v[i, :]

                def lanes(cc, carry3):
                    sl = pl.ds(cc * 16, 16)
                    out_v[i, sl] = (w0 * rows_v[2 * i, sl]
                                    + w1 * rows_v[2 * i + 1, sl])
                    return carry3

                lax.fori_loop(0, D // 16, lanes, 0)
                return carry2

            lax.fori_loop(0, _TCH, tok, 0)
            pltpu.sync_copy(out_v, out_hbm.at[pl.ds(t0, _TCH)])
            return carry

        lax.fori_loop(0, _TPW // _TCH, chunk, 0)

    return k(y, posflat, we16, wo16)


def kernel(inputs, Wr, br, W1, b1, W2, b2):
    x = inputs.reshape(S, D)
    idx, w = _router(x, Wr, br.reshape(1, E))

    a = idx.reshape(-1)
    onehot = (a[:, None] == jnp.arange(E, dtype=jnp.int32)).astype(jnp.int32)
    csum = jnp.cumsum(onehot, axis=0)
    counts = csum[-1]
    rank = jnp.take_along_axis(csum, a[:, None], axis=1)[:, 0] - 1
    padded = ((counts + BM - 1) // BM) * BM
    pad_end = jnp.cumsum(padded)
    pad_off = pad_end - padded
    pos = (pad_off[a] + rank).astype(jnp.int32)
    nact = (pad_end[-1] // BM).astype(jnp.int32)
    be = jnp.searchsorted(
        pad_end, jnp.arange(NB, dtype=jnp.int32) * BM,
        side="right").astype(jnp.int32)
    be = jnp.minimum(be, be[nact - 1])
    sp = jnp.concatenate([nact[None], be])

    pos2 = pos.reshape(S, K)
    we16 = jnp.broadcast_to(w[:, 0:1], (S, 16))
    wo16 = jnp.broadcast_to(w[:, 1:2], (S, 16))

    xp = lax.bitcast_convert_type(
        x.astype(jnp.bfloat16).reshape(S, D // 2, 2), jnp.int32)
    xsp = _sc_dispatch(xp, pos2[:, 0], pos2[:, 1])
    xs = lax.bitcast_convert_type(xsp, jnp.bfloat16).reshape(NR, D)
    y = _ffn(sp, xs, W1, b1, W2, b2)
    out = _sc_combine(y, pos, we16, wo16)
    return out.reshape(1, S, D)


# speedup vs baseline: 1.8173x; 1.8173x over previous
import functools

import jax
import jax.numpy as jnp
from jax import lax
from jax.experimental import pallas as pl
from jax.experimental.pallas import tpu as pltpu
from jax.experimental.pallas import tpu_sc as plsc

S, D, F, E, K = 2048, 1024, 4096, 8, 2
NA = S * K
BM = 256
NB = (NA + E * (BM - 1) + BM - 1) // BM
NR = NB * BM
FSPLIT = 2
FB = F // FSPLIT

_NC, _NS = 2, 16
_NW = _NC * _NS
_RPW = NR // _NW
_GCH = 64
_TPW = S // _NW
_TCH = 32


def _router_body(x_ref, wr_ref, br_ref, idx_ref, w_ref):
    x = x_ref[...]
    logits = lax.dot_general(x, wr_ref[...], (((1,), (0,)), ((), ())),
                             preferred_element_type=jnp.float32)
    logits = logits + br_ref[...]
    ids = lax.broadcasted_iota(jnp.int32, (S, E), 1)
    v1 = jnp.max(logits, axis=1, keepdims=True)
    i1 = jnp.min(jnp.where(logits == v1, ids, E), axis=1, keepdims=True)
    neg = jnp.float32(-3.4e38)
    l2 = jnp.where(ids == i1, neg, logits)
    v2 = jnp.max(l2, axis=1, keepdims=True)
    i2 = jnp.min(jnp.where(l2 == v2, ids, E), axis=1, keepdims=True)
    t = jnp.exp(v2 - v1)
    w1 = 1.0 / (1.0 + t)
    w2 = t / (1.0 + t)
    lane = lax.broadcasted_iota(jnp.int32, (S, K), 1)
    idx_ref[...] = jnp.where(lane == 0, i1, i2)
    w_ref[...] = jnp.where(lane == 0, w1, w2)


def _router(x, Wr, br2):
    return pl.pallas_call(
        _router_body,
        out_shape=(jax.ShapeDtypeStruct((S, K), jnp.int32),
                   jax.ShapeDtypeStruct((S, K), jnp.float32)),
    )(x, Wr, br2)


def _sc_dispatch(xp, pos_e, pos_o):
    mesh = plsc.VectorSubcoreMesh(core_axis_name="c", subcore_axis_name="s")

    @functools.partial(
        pl.kernel,
        out_type=jax.ShapeDtypeStruct((NR, D), jnp.float32),
        mesh=mesh,
        scratch_types=[pltpu.VMEM((_TPW,), jnp.int32),
                       pltpu.VMEM((_TPW,), jnp.int32),
                       pltpu.VMEM((_TPW, D), jnp.float32),
                       pltpu.SemaphoreType.DMA],
    )
    def k(x_hbm, pe_hbm, po_hbm, out_hbm, idxe_v, idxo_v, rows_v, sem):
        wid = lax.axis_index("s") * _NC + lax.axis_index("c")
        t0 = wid * _TPW
        pltpu.sync_copy(pe_hbm.at[pl.ds(t0, _TPW)], idxe_v)
        pltpu.sync_copy(po_hbm.at[pl.ds(t0, _TPW)], idxo_v)
        pltpu.sync_copy(x_hbm.at[pl.ds(t0, _TPW)], rows_v)
        cp_e = pltpu.async_copy(rows_v, out_hbm.at[idxe_v], sem)
        cp_o = pltpu.async_copy(rows_v, out_hbm.at[idxo_v], sem)
        cp_e.wait()
        cp_o.wait()

    return k(xp, pos_e, pos_o)


def _ffn1_body(s_ref, xs_ref, w1_ref, b1_ref, h_ref):
    b = pl.program_id(0)

    @pl.when(b < s_ref[0])
    def _():
        h = lax.dot_general(xs_ref[...], w1_ref[0], (((1,), (0,)), ((), ())),
                            preferred_element_type=jnp.float32)
        h = jnp.maximum(h + b1_ref[0], 0.0)
        h_ref[...] = h.astype(jnp.bfloat16)


def _ffn1(sp, xs, W1, b1):
    grid_spec = pltpu.PrefetchScalarGridSpec(
        num_scalar_prefetch=1,
        grid=(NB,),
        in_specs=[
            pl.BlockSpec((BM, D), lambda b, s: (b, 0)),
            pl.BlockSpec((1, D, F), lambda b, s: (s[1 + b], 0, 0)),
            pl.BlockSpec((1, 1, F), lambda b, s: (s[1 + b], 0, 0)),
        ],
        out_specs=pl.BlockSpec((BM, F), lambda b, s: (b, 0)),
    )
    return pl.pallas_call(
        _ffn1_body,
        grid_spec=grid_spec,
        out_shape=jax.ShapeDtypeStruct((NR, F), jnp.bfloat16),
        compiler_params=pltpu.CompilerParams(
            dimension_semantics=("arbitrary",)),
    )(sp, xs, W1, b1.reshape(E, 1, F))


def _ffn2_body(s_ref, h_ref, w2_ref, b2_ref, y_ref):
    b = pl.program_id(0)

    @pl.when(b < s_ref[0])
    def _():
        hb = h_ref[...].astype(jnp.float32)
        y = lax.dot_general(hb, w2_ref[0], (((1,), (0,)), ((), ())),
                            preferred_element_type=jnp.float32)
        y_ref[...] = y + b2_ref[0]


def _ffn2(sp, h, W2, b2):
    grid_spec = pltpu.PrefetchScalarGridSpec(
        num_scalar_prefetch=1,
        grid=(NB,),
        in_specs=[
            pl.BlockSpec((BM, F), lambda b, s: (b, 0)),
            pl.BlockSpec((1, F, D), lambda b, s: (s[1 + b], 0, 0)),
            pl.BlockSpec((1, 1, D), lambda b, s: (s[1 + b], 0, 0)),
        ],
        out_specs=pl.BlockSpec((BM, D), lambda b, s: (b, 0)),
    )
    return pl.pallas_call(
        _ffn2_body,
        grid_spec=grid_spec,
        out_shape=jax.ShapeDtypeStruct((NR, D), jnp.float32),
        compiler_params=pltpu.CompilerParams(
            dimension_semantics=("arbitrary",)),
    )(sp, h, W2, b2.reshape(E, 1, D))


def _sc_combine(y, posflat, we16, wo16):
    mesh = plsc.VectorSubcoreMesh(core_axis_name="c", subcore_axis_name="s")

    @functools.partial(
        pl.kernel,
        out_type=jax.ShapeDtypeStruct((S, D), jnp.float32),
        mesh=mesh,
        scratch_types=[pltpu.VMEM((K * _TCH,), jnp.int32),
                       pltpu.VMEM((_TCH, 16), jnp.float32),
                       pltpu.VMEM((_TCH, 16), jnp.float32),
                       pltpu.VMEM((K * _TCH, D), jnp.float32),
                       pltpu.VMEM((_TCH, D), jnp.float32),
                       pltpu.SemaphoreType.DMA],
    )
    def k(y_hbm, pos_hbm, we_hbm, wo_hbm, out_hbm,
          idx_v, we_v, wo_v, rows_v, out_v, sem):
        wid = lax.axis_index("s") * _NC + lax.axis_index("c")
        tbase = wid * _TPW

        def chunk(c, carry):
            t0 = tbase + c * _TCH
            pltpu.sync_copy(pos_hbm.at[pl.ds(t0 * K, K * _TCH)], idx_v)
            pltpu.sync_copy(we_hbm.at[pl.ds(t0, _TCH)], we_v)
            pltpu.sync_copy(wo_hbm.at[pl.ds(t0, _TCH)], wo_v)
            pltpu.async_copy(y_hbm.at[idx_v], rows_v, sem).wait()

            def tok(i, carry2):
                w0 = we_v[i, :]
                w1 = wo_v[i, :]

                def lanes(cc, carry3):
                    sl = pl.ds(cc * 16, 16)
                    out_v[i, sl] = (w0 * rows_v[2 * i, sl]
                                    + w1 * rows_v[2 * i + 1, sl])
                    return carry3

                lax.fori_loop(0, D // 16, lanes, 0)
                return carry2

            lax.fori_loop(0, _TCH, tok, 0)
            pltpu.sync_copy(out_v, out_hbm.at[pl.ds(t0, _TCH)])
            return carry

        lax.fori_loop(0, _TPW // _TCH, chunk, 0)

    return k(y, posflat, we16, wo16)


def kernel(inputs, Wr, br, W1, b1, W2, b2):
    x = inputs.reshape(S, D)
    idx, w = _router(x, Wr, br.reshape(1, E))

    a = idx.reshape(-1)
    onehot = (a[:, None] == jnp.arange(E, dtype=jnp.int32)).astype(jnp.int32)
    csum = jnp.cumsum(onehot, axis=0)
    counts = csum[-1]
    rank = jnp.take_along_axis(csum, a[:, None], axis=1)[:, 0] - 1
    padded = ((counts + BM - 1) // BM) * BM
    pad_end = jnp.cumsum(padded)
    pad_off = pad_end - padded
    pos = (pad_off[a] + rank).astype(jnp.int32)
    nact = (pad_end[-1] // BM).astype(jnp.int32)
    be = jnp.searchsorted(
        pad_end, jnp.arange(NB, dtype=jnp.int32) * BM,
        side="right").astype(jnp.int32)
    be = jnp.minimum(be, be[nact - 1])
    sp = jnp.concatenate([nact[None], be])

    pos2 = pos.reshape(S, K)
    we16 = jnp.broadcast_to(w[:, 0:1], (S, 16))
    wo16 = jnp.broadcast_to(w[:, 1:2], (S, 16))

    xs = _sc_dispatch(x, pos2[:, 0], pos2[:, 1])
    h = _ffn1(sp, xs, W1, b1)
    y = _ffn2(sp, h, W2, b2)
    out = _sc_combine(y, pos, we16, wo16)
    return out.reshape(1, S, D)


# speedup vs baseline: 1.9386x; 1.0667x over previous
import functools

import jax
import jax.numpy as jnp
from jax import lax
from jax.experimental import pallas as pl
from jax.experimental.pallas import tpu as pltpu
from jax.experimental.pallas import tpu_sc as plsc

S, D, F, E, K = 2048, 1024, 4096, 8, 2
NA = S * K
BM = 512
NB = (NA + E * (BM - 1) + BM - 1) // BM
NR = NB * BM
FSPLIT = 2
FB = F // FSPLIT

_NC, _NS = 2, 16
_NW = _NC * _NS
_RPW = NR // _NW
_GCH = 64
_TPW = S // _NW
_TCH = 32


def _router_body(x_ref, wr_ref, br_ref, idx_ref, w_ref):
    x = x_ref[...]
    logits = lax.dot_general(x, wr_ref[...], (((1,), (0,)), ((), ())),
                             preferred_element_type=jnp.float32)
    logits = logits + br_ref[...]
    ids = lax.broadcasted_iota(jnp.int32, (S, E), 1)
    v1 = jnp.max(logits, axis=1, keepdims=True)
    i1 = jnp.min(jnp.where(logits == v1, ids, E), axis=1, keepdims=True)
    neg = jnp.float32(-3.4e38)
    l2 = jnp.where(ids == i1, neg, logits)
    v2 = jnp.max(l2, axis=1, keepdims=True)
    i2 = jnp.min(jnp.where(l2 == v2, ids, E), axis=1, keepdims=True)
    t = jnp.exp(v2 - v1)
    w1 = 1.0 / (1.0 + t)
    w2 = t / (1.0 + t)
    lane = lax.broadcasted_iota(jnp.int32, (S, K), 1)
    idx_ref[...] = jnp.where(lane == 0, i1, i2)
    w_ref[...] = jnp.where(lane == 0, w1, w2)


def _router(x, Wr, br2):
    return pl.pallas_call(
        _router_body,
        out_shape=(jax.ShapeDtypeStruct((S, K), jnp.int32),
                   jax.ShapeDtypeStruct((S, K), jnp.float32)),
    )(x, Wr, br2)


def _sc_dispatch(xp, pos_e, pos_o):
    mesh = plsc.VectorSubcoreMesh(core_axis_name="c", subcore_axis_name="s")

    @functools.partial(
        pl.kernel,
        out_type=jax.ShapeDtypeStruct((NR, D), jnp.float32),
        mesh=mesh,
        scratch_types=[pltpu.VMEM((_TPW,), jnp.int32),
                       pltpu.VMEM((_TPW,), jnp.int32),
                       pltpu.VMEM((_TPW, D), jnp.float32),
                       pltpu.SemaphoreType.DMA],
    )
    def k(x_hbm, pe_hbm, po_hbm, out_hbm, idxe_v, idxo_v, rows_v, sem):
        wid = lax.axis_index("s") * _NC + lax.axis_index("c")
        t0 = wid * _TPW
        pltpu.sync_copy(pe_hbm.at[pl.ds(t0, _TPW)], idxe_v)
        pltpu.sync_copy(po_hbm.at[pl.ds(t0, _TPW)], idxo_v)
        pltpu.sync_copy(x_hbm.at[pl.ds(t0, _TPW)], rows_v)
        cp_e = pltpu.async_copy(rows_v, out_hbm.at[idxe_v], sem)
        cp_o = pltpu.async_copy(rows_v, out_hbm.at[idxo_v], sem)
        cp_e.wait()
        cp_o.wait()

    return k(xp, pos_e, pos_o)


def _ffn1_body(s_ref, xs_ref, w1_ref, b1_ref, h_ref):
    b = pl.program_id(0)

    @pl.when(b < s_ref[0])
    def _():
        h = lax.dot_general(xs_ref[...], w1_ref[0], (((1,), (0,)), ((), ())),
                            preferred_element_type=jnp.float32)
        h = jnp.maximum(h + b1_ref[0], 0.0)
        h_ref[...] = h.astype(jnp.bfloat16)


def _ffn1(sp, xs, W1, b1):
    grid_spec = pltpu.PrefetchScalarGridSpec(
        num_scalar_prefetch=1,
        grid=(NB,),
        in_specs=[
            pl.BlockSpec((BM, D), lambda b, s: (b, 0)),
            pl.BlockSpec((1, D, F), lambda b, s: (s[1 + b], 0, 0)),
            pl.BlockSpec((1, 1, F), lambda b, s: (s[1 + b], 0, 0)),
        ],
        out_specs=pl.BlockSpec((BM, F), lambda b, s: (b, 0)),
    )
    return pl.pallas_call(
        _ffn1_body,
        grid_spec=grid_spec,
        out_shape=jax.ShapeDtypeStruct((NR, F), jnp.bfloat16),
        compiler_params=pltpu.CompilerParams(
            dimension_semantics=("arbitrary",)),
    )(sp, xs, W1, b1.reshape(E, 1, F))


def _ffn2_body(s_ref, h_ref, w2_ref, b2_ref, y_ref):
    b = pl.program_id(0)

    @pl.when(b < s_ref[0])
    def _():
        hb = h_ref[...].astype(jnp.float32)
        y = lax.dot_general(hb, w2_ref[0], (((1,), (0,)), ((), ())),
                            preferred_element_type=jnp.float32)
        y_ref[...] = y + b2_ref[0]


def _ffn2(sp, h, W2, b2):
    grid_spec = pltpu.PrefetchScalarGridSpec(
        num_scalar_prefetch=1,
        grid=(NB,),
        in_specs=[
            pl.BlockSpec((BM, F), lambda b, s: (b, 0)),
            pl.BlockSpec((1, F, D), lambda b, s: (s[1 + b], 0, 0)),
            pl.BlockSpec((1, 1, D), lambda b, s: (s[1 + b], 0, 0)),
        ],
        out_specs=pl.BlockSpec((BM, D), lambda b, s: (b, 0)),
    )
    return pl.pallas_call(
        _ffn2_body,
        grid_spec=grid_spec,
        out_shape=jax.ShapeDtypeStruct((NR, D), jnp.float32),
        compiler_params=pltpu.CompilerParams(
            dimension_semantics=("arbitrary",)),
    )(sp, h, W2, b2.reshape(E, 1, D))


def _sc_combine(y, posflat, we16, wo16):
    mesh = plsc.VectorSubcoreMesh(core_axis_name="c", subcore_axis_name="s")

    @functools.partial(
        pl.kernel,
        out_type=jax.ShapeDtypeStruct((S, D), jnp.float32),
        mesh=mesh,
        scratch_types=[pltpu.VMEM((K * _TCH,), jnp.int32),
                       pltpu.VMEM((_TCH, 16), jnp.float32),
                       pltpu.VMEM((_TCH, 16), jnp.float32),
                       pltpu.VMEM((K * _TCH, D), jnp.float32),
                       pltpu.VMEM((_TCH, D), jnp.float32),
                       pltpu.SemaphoreType.DMA],
    )
    def k(y_hbm, pos_hbm, we_hbm, wo_hbm, out_hbm,
          idx_v, we_v, wo_v, rows_v, out_v, sem):
        wid = lax.axis_index("s") * _NC + lax.axis_index("c")
        tbase = wid * _TPW

        def chunk(c, carry):
            t0 = tbase + c * _TCH
            pltpu.sync_copy(pos_hbm.at[pl.ds(t0 * K, K * _TCH)], idx_v)
            pltpu.sync_copy(we_hbm.at[pl.ds(t0, _TCH)], we_v)
            pltpu.sync_copy(wo_hbm.at[pl.ds(t0, _TCH)], wo_v)
            pltpu.async_copy(y_hbm.at[idx_v], rows_v, sem).wait()

            def tok(i, carry2):
                w0 = we_v[i, :]
                w1 = wo_v[i, :]

                def lanes(cc, carry3):
                    for u in range(4):
                        sl = pl.ds(cc * 64 + u * 16, 16)
                        out_v[i, sl] = (w0 * rows_v[2 * i, sl]
                                        + w1 * rows_v[2 * i + 1, sl])
                    return carry3

                lax.fori_loop(0, D // 64, lanes, 0)
                return carry2

            lax.fori_loop(0, _TCH, tok, 0)
            pltpu.sync_copy(out_v, out_hbm.at[pl.ds(t0, _TCH)])
            return carry

        lax.fori_loop(0, _TPW // _TCH, chunk, 0)

    return k(y, posflat, we16, wo16)


def kernel(inputs, Wr, br, W1, b1, W2, b2):
    x = inputs.reshape(S, D)
    idx, w = _router(x, Wr, br.reshape(1, E))

    a = idx.reshape(-1)
    oh = (a[None, :] == jnp.arange(E, dtype=jnp.int32)[:, None]).astype(
        jnp.int32)
    csum = jnp.cumsum(oh, axis=1)
    counts = csum[:, -1]
    rank = jnp.sum(oh * csum, axis=0) - 1
    padded = ((counts + BM - 1) // BM) * BM
    pad_end = jnp.cumsum(padded)
    pad_off = pad_end - padded
    pos = (jnp.sum(oh * pad_off[:, None], axis=0) + rank).astype(jnp.int32)
    nact = (pad_end[-1] // BM).astype(jnp.int32)
    be = jnp.sum(
        pad_end[None, :] <= (jnp.arange(NB, dtype=jnp.int32) * BM)[:, None],
        axis=1).astype(jnp.int32)
    last_e = jnp.max(jnp.where(padded > 0, jnp.arange(E, dtype=jnp.int32), 0))
    be = jnp.minimum(be, last_e)
    sp = jnp.concatenate([nact[None], be])

    pos2 = pos.reshape(S, K)
    we16 = jnp.broadcast_to(w[:, 0:1], (S, 16))
    wo16 = jnp.broadcast_to(w[:, 1:2], (S, 16))

    xs = _sc_dispatch(x, pos2[:, 0], pos2[:, 1])
    h = _ffn1(sp, xs, W1, b1)
    y = _ffn2(sp, h, W2, b2)
    out = _sc_combine(y, pos, we16, wo16)
    return out.reshape(1, S, D)


# speedup vs baseline: 2.1199x; 1.0935x over previous
import functools

import jax
import jax.numpy as jnp
from jax import lax
from jax.experimental import pallas as pl
from jax.experimental.pallas import tpu as pltpu
from jax.experimental.pallas import tpu_sc as plsc

S, D, F, E, K = 2048, 1024, 4096, 8, 2
NA = S * K
BM = 512
NB = (NA + E * (BM - 1) + BM - 1) // BM
NR = NB * BM
FSPLIT = 2
FB = F // FSPLIT

_NC, _NS = 2, 16
_NW = _NC * _NS
_RPW = NR // _NW
_GCH = 64
_TPW = S // _NW
_TCH = 16
_NCH = _TPW // _TCH


def _router_body(x_ref, wr_ref, br_ref, idx_ref, we_ref, wo_ref):
    x = x_ref[...]
    logits = lax.dot_general(x, wr_ref[...], (((1,), (0,)), ((), ())),
                             preferred_element_type=jnp.float32)
    logits = logits + br_ref[...]
    ids = lax.broadcasted_iota(jnp.int32, (S, E), 1)
    v1 = jnp.max(logits, axis=1, keepdims=True)
    i1 = jnp.min(jnp.where(logits == v1, ids, E), axis=1, keepdims=True)
    neg = jnp.float32(-3.4e38)
    l2 = jnp.where(ids == i1, neg, logits)
    v2 = jnp.max(l2, axis=1, keepdims=True)
    i2 = jnp.min(jnp.where(l2 == v2, ids, E), axis=1, keepdims=True)
    t = jnp.exp(v2 - v1)
    w1 = 1.0 / (1.0 + t)
    w2 = t / (1.0 + t)
    lane = lax.broadcasted_iota(jnp.int32, (S, K), 1)
    idx_ref[...] = jnp.where(lane == 0, i1, i2)
    we_ref[...] = jnp.broadcast_to(w1, (S, 16))
    wo_ref[...] = jnp.broadcast_to(w2, (S, 16))


def _router(x, Wr, br2):
    return pl.pallas_call(
        _router_body,
        out_shape=(jax.ShapeDtypeStruct((S, K), jnp.int32),
                   jax.ShapeDtypeStruct((S, 16), jnp.float32),
                   jax.ShapeDtypeStruct((S, 16), jnp.float32)),
    )(x, Wr, br2)


def _sc_dispatch(xp, pos_e, pos_o):
    mesh = plsc.VectorSubcoreMesh(core_axis_name="c", subcore_axis_name="s")

    @functools.partial(
        pl.kernel,
        out_type=jax.ShapeDtypeStruct((NR, D), jnp.float32),
        mesh=mesh,
        scratch_types=[pltpu.VMEM((_TPW,), jnp.int32),
                       pltpu.VMEM((_TPW,), jnp.int32),
                       pltpu.VMEM((_TPW, D), jnp.float32),
                       pltpu.SemaphoreType.DMA],
    )
    def k(x_hbm, pe_hbm, po_hbm, out_hbm, idxe_v, idxo_v, rows_v, sem):
        wid = lax.axis_index("s") * _NC + lax.axis_index("c")
        t0 = wid * _TPW
        pltpu.sync_copy(pe_hbm.at[pl.ds(t0, _TPW)], idxe_v)
        pltpu.sync_copy(po_hbm.at[pl.ds(t0, _TPW)], idxo_v)
        pltpu.sync_copy(x_hbm.at[pl.ds(t0, _TPW)], rows_v)
        cp_e = pltpu.async_copy(rows_v, out_hbm.at[idxe_v], sem)
        cp_o = pltpu.async_copy(rows_v, out_hbm.at[idxo_v], sem)
        cp_e.wait()
        cp_o.wait()

    return k(xp, pos_e, pos_o)


def _ffn1_body(s_ref, xs_ref, w1_ref, b1_ref, h_ref):
    b = pl.program_id(0)

    @pl.when(b < s_ref[0])
    def _():
        h = lax.dot_general(xs_ref[...], w1_ref[0], (((1,), (0,)), ((), ())),
                            preferred_element_type=jnp.float32)
        h = jnp.maximum(h + b1_ref[0], 0.0)
        h_ref[...] = h.astype(jnp.bfloat16)


def _ffn1(sp, xs, W1, b1):
    grid_spec = pltpu.PrefetchScalarGridSpec(
        num_scalar_prefetch=1,
        grid=(NB,),
        in_specs=[
            pl.BlockSpec((BM, D),
                         lambda b, s: (jnp.minimum(b, s[0] - 1), 0)),
            pl.BlockSpec((1, D, F), lambda b, s: (s[1 + b], 0, 0)),
            pl.BlockSpec((1, 1, F), lambda b, s: (s[1 + b], 0, 0)),
        ],
        out_specs=pl.BlockSpec((BM, F),
                               lambda b, s: (jnp.minimum(b, s[0] - 1), 0)),
    )
    return pl.pallas_call(
        _ffn1_body,
        grid_spec=grid_spec,
        out_shape=jax.ShapeDtypeStruct((NR, F), jnp.bfloat16),
        compiler_params=pltpu.CompilerParams(
            dimension_semantics=("arbitrary",)),
    )(sp, xs, W1, b1.reshape(E, 1, F))


def _ffn2_body(s_ref, h_ref, w2_ref, b2_ref, y_ref):
    b = pl.program_id(0)

    @pl.when(b < s_ref[0])
    def _():
        hb = h_ref[...].astype(jnp.float32)
        y = lax.dot_general(hb, w2_ref[0], (((1,), (0,)), ((), ())),
                            preferred_element_type=jnp.float32)
        y_ref[...] = y + b2_ref[0]


def _ffn2(sp, h, W2, b2):
    grid_spec = pltpu.PrefetchScalarGridSpec(
        num_scalar_prefetch=1,
        grid=(NB,),
        in_specs=[
            pl.BlockSpec((BM, F),
                         lambda b, s: (jnp.minimum(b, s[0] - 1), 0)),
            pl.BlockSpec((1, F, D), lambda b, s: (s[1 + b], 0, 0)),
            pl.BlockSpec((1, 1, D), lambda b, s: (s[1 + b], 0, 0)),
        ],
        out_specs=pl.BlockSpec((BM, D),
                               lambda b, s: (jnp.minimum(b, s[0] - 1), 0)),
    )
    return pl.pallas_call(
        _ffn2_body,
        grid_spec=grid_spec,
        out_shape=jax.ShapeDtypeStruct((NR, D), jnp.float32),
        compiler_params=pltpu.CompilerParams(
            dimension_semantics=("arbitrary",)),
    )(sp, h, W2, b2.reshape(E, 1, D))


def _sc_combine(y, posflat, we16, wo16):
    mesh = plsc.VectorSubcoreMesh(core_axis_name="c", subcore_axis_name="s")

    @functools.partial(
        pl.kernel,
        out_type=jax.ShapeDtypeStruct((S, D), jnp.float32),
        mesh=mesh,
        scratch_types=[pltpu.VMEM((K * _TPW,), jnp.int32),
                       pltpu.VMEM((_TPW, 16), jnp.float32),
                       pltpu.VMEM((_TPW, 16), jnp.float32),
                       pltpu.VMEM((2, K * _TCH, D), jnp.float32),
                       pltpu.VMEM((2, _TCH, D), jnp.float32),
                       pltpu.SemaphoreType.DMA],
    )
    def k(y_hbm, pos_hbm, we_hbm, wo_hbm, out_hbm,
          idx_v, we_v, wo_v, rows_v, out_v, gsem):
        wid = lax.axis_index("s") * _NC + lax.axis_index("c")
        tbase = wid * _TPW
        pltpu.sync_copy(pos_hbm.at[pl.ds(tbase * K, K * _TPW)], idx_v)
        pltpu.sync_copy(we_hbm.at[pl.ds(tbase, _TPW)], we_v)
        pltpu.sync_copy(wo_hbm.at[pl.ds(tbase, _TPW)], wo_v)
        pltpu.async_copy(y_hbm.at[idx_v.at[pl.ds(0, K * _TCH)]],
                         rows_v.at[0], gsem)

        def chunk(c, carry):
            slot = lax.rem(c, 2)
            pltpu.make_async_copy(y_hbm.at[idx_v.at[pl.ds(0, K * _TCH)]],
                                  rows_v.at[0], gsem).wait()

            @pl.when(c + 1 < _NCH)
            def _():
                nslot = lax.rem(c + 1, 2)
                pltpu.async_copy(
                    y_hbm.at[idx_v.at[pl.ds((c + 1) * K * _TCH, K * _TCH)]],
                    rows_v.at[nslot], gsem)

            def tok(i, carry2):
                ti = c * _TCH + i
                w0 = we_v[ti, :]
                w1 = wo_v[ti, :]

                def lanes(cc, carry3):
                    for u in range(4):
                        sl = pl.ds(cc * 64 + u * 16, 16)
                        out_v[slot, i, sl] = (
                            w0 * rows_v[slot, 2 * i, sl]
                            + w1 * rows_v[slot, 2 * i + 1, sl])
                    return carry3

                lax.fori_loop(0, D // 64, lanes, 0)
                return carry2

            lax.fori_loop(0, _TCH, tok, 0)
            pltpu.sync_copy(out_v.at[slot],
                            out_hbm.at[pl.ds(tbase + c * _TCH, _TCH)])
            return carry

        lax.fori_loop(0, _NCH, chunk, 0)

    return k(y, posflat, we16, wo16)


def kernel(inputs, Wr, br, W1, b1, W2, b2):
    x = inputs.reshape(S, D)
    idx, we16, wo16 = _router(x, Wr, br.reshape(1, E))

    a = idx.reshape(-1)
    oh = (a[None, :] == jnp.arange(E, dtype=jnp.int32)[:, None]).astype(
        jnp.int32)
    csum = jnp.cumsum(oh, axis=1)
    counts = csum[:, -1]
    rank = jnp.sum(oh * csum, axis=0) - 1
    padded = ((counts + BM - 1) // BM) * BM
    pad_end = jnp.cumsum(padded)
    pad_off = pad_end - padded
    pos = (jnp.sum(oh * pad_off[:, None], axis=0) + rank).astype(jnp.int32)
    nact = (pad_end[-1] // BM).astype(jnp.int32)
    be = jnp.sum(
        pad_end[None, :] <= (jnp.arange(NB, dtype=jnp.int32) * BM)[:, None],
        axis=1).astype(jnp.int32)
    last_e = jnp.max(jnp.where(padded > 0, jnp.arange(E, dtype=jnp.int32), 0))
    be = jnp.minimum(be, last_e)
    sp = jnp.concatenate([nact[None], be])

    pos2 = pos.reshape(S, K)

    xs = _sc_dispatch(x, pos2[:, 0], pos2[:, 1])
    h = _ffn1(sp, xs, W1, b1)
    y = _ffn2(sp, h, W2, b2)
    out = _sc_combine(y, pos, we16, wo16)
    return out.reshape(1, S, D)
